# baseline (speedup 1.0000x reference)
import functools

import jax
import jax.numpy as jnp
from jax import lax
from jax.experimental import pallas as pl
from jax.experimental.pallas import tpu as pltpu
from jax.experimental.pallas import tpu_sc as plsc

N = 10000
E = 320000
D = 128

NC, NS = 2, 16
NW = NC * NS
BATCH = 128
NBLK = 80
EPT = NBLK * BATCH
EP = NW * EPT
RPT = 640
NPAD = NS * RPT
ZROWS = 80

_mesh = plsc.VectorSubcoreMesh(
    core_axis_name="c", subcore_axis_name="s", num_cores=NC, num_subcores=NS
)

def _deg_body(cidx_hbm, out_hbm, cidx, ones_v, zbuf, dacc):
    c = lax.axis_index("c")
    s = lax.axis_index("s")
    wid = c * NS + s
    _ones16 = jnp.ones((16,), jnp.float32)
    _zeros16 = jnp.zeros((16,), jnp.float32)

    def fill_ones(i, _):
        ones_v[pl.ds(i * 16, 16)] = _ones16
        return 0

    lax.fori_loop(0, BATCH // 16, fill_ones, 0)

    def fill_zero(i, _):
        zbuf[pl.ds(i * 16, 16)] = _zeros16
        return 0

    lax.fori_loop(0, RPT // 16, fill_zero, 0)

    pltpu.sync_copy(zbuf, dacc.at[pl.ds(s * RPT, RPT)])
    plsc.subcore_barrier()

    pltpu.sync_copy(cidx_hbm.at[wid], cidx)

    def blk(j, _):
        pltpu.sync_copy(ones_v, dacc.at[cidx.at[j]], add=True)
        return 0

    lax.fori_loop(0, NBLK, blk, 0)
    plsc.subcore_barrier()
    pltpu.sync_copy(dacc.at[pl.ds(s * RPT, RPT)], out_hbm.at[c, pl.ds(s * RPT, RPT)])


_deg = pl.kernel(
    _deg_body,
    out_type=jax.ShapeDtypeStruct((NC, NPAD), jnp.float32),
    mesh=_mesh,
    scratch_types=[
        pltpu.VMEM((NBLK, BATCH), jnp.int32),
        pltpu.VMEM((BATCH,), jnp.float32),
        pltpu.VMEM((RPT,), jnp.float32),
        pltpu.VMEM_SHARED((NPAD,), jnp.float32),
    ],
)


EPB = EP // BATCH
B0 = 160
B1 = 0
CH = 40


def _seg_body(
    g_hbm, ridx_hbm, cidx_hbm, out_hbm, ridx, cidx, rows0, rows1, acc, sem0, sem1
):
    c = lax.axis_index("c")
    s = lax.axis_index("s")
    _zeros16 = jnp.zeros((16,), jnp.float32)

    def fill_zero(i, _):
        def lane(k, _):
            rows0[i, pl.ds(k * 16, 16)] = _zeros16
            return 0

        lax.fori_loop(0, D // 16, lane, 0)
        return 0

    lax.fori_loop(0, BATCH, fill_zero, 0)

    def zcopy(k, _):
        pltpu.sync_copy(rows0, acc.at[pl.ds(s * RPT + k * BATCH, BATCH), :])
        return 0

    lax.fori_loop(0, RPT // BATCH, zcopy, 0)
    plsc.subcore_barrier()

    rows = (rows0, rows1)
    sems = (sem0, sem1)

    start_blk = jnp.where(c == 0, s * B0, NS * B0 + s * B1)
    nstage = jnp.where(c == 0, B0 // CH, B1 // CH)

    def stage(st, _):
        base = start_blk + st * CH
        pltpu.sync_copy(ridx_hbm.at[pl.ds(base, CH), :], ridx)
        pltpu.sync_copy(cidx_hbm.at[pl.ds(base, CH), :], cidx)

        pltpu.async_copy(g_hbm.at[ridx.at[0]], rows0, sem0)
        pltpu.async_copy(g_hbm.at[ridx.at[1]], rows1, sem1)

        def blk(i, _):
            for b in range(2):
                j = i * 2 + b
                pltpu.make_async_copy(
                    g_hbm.at[ridx.at[j]], rows[b], sems[b]
                ).wait()
                pltpu.sync_copy(rows[b], acc.at[cidx.at[j]], add=True)

                @pl.when(j + 2 < CH)
                def _():
                    pltpu.async_copy(g_hbm.at[ridx.at[j + 2]], rows[b], sems[b])

            return 0

        lax.fori_loop(0, CH // 2, blk, 0)
        return 0

    lax.fori_loop(0, nstage, stage, 0)

    plsc.subcore_barrier()
    pltpu.sync_copy(
        acc.at[pl.ds(s * RPT, RPT), :], out_hbm.at[c, pl.ds(s * RPT, RPT), :]
    )


_seg = pl.kernel(
    _seg_body,
    out_type=jax.ShapeDtypeStruct((NC, NPAD, D), jnp.float32),
    mesh=_mesh,
    scratch_types=[
        pltpu.VMEM((CH, BATCH), jnp.int32),
        pltpu.VMEM((CH, BATCH), jnp.int32),
        pltpu.VMEM((BATCH, D), jnp.float32),
        pltpu.VMEM((BATCH, D), jnp.float32),
        pltpu.VMEM_SHARED((NPAD, D), jnp.float32),
        pltpu.SemaphoreType.DMA,
        pltpu.SemaphoreType.DMA,
    ],
)


BR = 1000


def _dinv(dega_ref, degb_ref):
    return lax.rsqrt(dega_ref[:, 0:1] + degb_ref[:, 0:1] + 1.0)


def _mm_scale_body(dega_ref, degb_ref, x_ref, w_ref, g_ref):
    dinv = _dinv(dega_ref, degb_ref)
    h = jnp.dot(x_ref[...], w_ref[...], preferred_element_type=jnp.float32)
    g_ref[...] = h * dinv


def _mid_body(dega_ref, degb_ref, sa_ref, sb_ref, g_ref, b_ref, w_ref, o_ref):
    dinv = _dinv(dega_ref, degb_ref)
    h = (sa_ref[...] + sb_ref[...] + g_ref[...]) * dinv + b_ref[...]
    h = jnp.maximum(h, 0.0)
    o_ref[...] = jnp.dot(h, w_ref[...], preferred_element_type=jnp.float32) * dinv


def _out_body(dega_ref, degb_ref, sa_ref, sb_ref, g_ref, b_ref, o_ref):
    dinv = _dinv(dega_ref, degb_ref)
    o_ref[...] = jnp.maximum(
        (sa_ref[...] + sb_ref[...] + g_ref[...]) * dinv + b_ref[...], 0.0
    )


_deg_spec = pl.BlockSpec((BR, 1), lambda i: (i, 0))
_row_spec = pl.BlockSpec((BR, D), lambda i: (i, 0))
_w_spec = pl.BlockSpec((D, D), lambda i: (0, 0))
_b_spec = pl.BlockSpec((1, D), lambda i: (0, 0))
_o_shape = jax.ShapeDtypeStruct((N, D), jnp.float32)

_mm_scale = pl.pallas_call(
    _mm_scale_body,
    grid=(N // BR,),
    in_specs=[_deg_spec, _deg_spec, _row_spec, _w_spec],
    out_specs=_row_spec,
    out_shape=_o_shape,
)

_mid = pl.pallas_call(
    _mid_body,
    grid=(N // BR,),
    in_specs=[_deg_spec, _deg_spec, _row_spec, _row_spec, _row_spec, _b_spec, _w_spec],
    out_specs=_row_spec,
    out_shape=_o_shape,
)

_out = pl.pallas_call(
    _out_body,
    grid=(N // BR,),
    in_specs=[_deg_spec, _deg_spec, _row_spec, _row_spec, _row_spec, _b_spec],
    out_specs=_row_spec,
    out_shape=_o_shape,
)


def kernel(x, edge_index, W1, b1, W2, b2):
    row = edge_index[0]
    col = edge_index[1]
    pad = EP - E
    rowp = jnp.concatenate([row, jnp.zeros((pad,), jnp.int32)]).reshape(
        NW, NBLK, BATCH
    )
    colp = jnp.concatenate([col, jnp.full((pad,), N, jnp.int32)]).reshape(
        NW, NBLK, BATCH
    )

    degp = _deg(colp)
    dega = degp[0, :N].reshape(N, 1)
    degb = degp[1, :N].reshape(N, 1)
    b1r = b1.reshape(1, D)
    b2r = b2.reshape(1, D)

    rowf = rowp.reshape(EPB, BATCH)
    colf = colp.reshape(EPB, BATCH)
    g1 = _mm_scale(dega, degb, x, W1)
    s1 = _seg(g1, rowf, colf)
    g2 = _mid(dega, degb, s1[0, :N], s1[1, :N], g1, b1r, W2)
    s2 = _seg(g2, rowf, colf)
    return _out(dega, degb, s2[0, :N], s2[1, :N], g2, b2r)


# speedup vs baseline: 1.1731x; 1.1731x over previous
import functools

import jax
import jax.numpy as jnp
from jax import lax
from jax.experimental import pallas as pl
from jax.experimental.pallas import tpu as pltpu
from jax.experimental.pallas import tpu_sc as plsc

N = 10000
E = 320000
D = 128

NC, NS = 2, 16
NW = NC * NS
BATCH = 128
NBLK = 80
EPT = NBLK * BATCH
EP = NW * EPT
RPT = 640
NPAD = NS * RPT
ZROWS = 80

_mesh = plsc.VectorSubcoreMesh(
    core_axis_name="c", subcore_axis_name="s", num_cores=NC, num_subcores=NS
)

def _deg_body(cidx_hbm, out_hbm, cidx, ones_v, zbuf, dacc):
    c = lax.axis_index("c")
    s = lax.axis_index("s")
    wid = c * NS + s
    _ones16 = jnp.ones((16,), jnp.float32)
    _zeros16 = jnp.zeros((16,), jnp.float32)

    def fill_ones(i, _):
        ones_v[pl.ds(i * 16, 16)] = _ones16
        return 0

    lax.fori_loop(0, BATCH // 16, fill_ones, 0)

    def fill_zero(i, _):
        zbuf[pl.ds(i * 16, 16)] = _zeros16
        return 0

    lax.fori_loop(0, RPT // 16, fill_zero, 0)

    pltpu.sync_copy(zbuf, dacc.at[pl.ds(s * RPT, RPT)])
    plsc.subcore_barrier()

    pltpu.sync_copy(cidx_hbm.at[wid], cidx)

    def blk(j, _):
        pltpu.sync_copy(ones_v, dacc.at[cidx.at[j]], add=True)
        return 0

    lax.fori_loop(0, NBLK, blk, 0)
    plsc.subcore_barrier()
    pltpu.sync_copy(dacc.at[pl.ds(s * RPT, RPT)], out_hbm.at[c, pl.ds(s * RPT, RPT)])


_deg = pl.kernel(
    _deg_body,
    out_type=jax.ShapeDtypeStruct((NC, NPAD), jnp.float32),
    mesh=_mesh,
    scratch_types=[
        pltpu.VMEM((NBLK, BATCH), jnp.int32),
        pltpu.VMEM((BATCH,), jnp.float32),
        pltpu.VMEM((RPT,), jnp.float32),
        pltpu.VMEM_SHARED((NPAD,), jnp.float32),
    ],
)


EPB = EP // BATCH
B0 = 120
B1 = 40
CH = 40


def _seg_body(
    g_hbm, ridx_hbm, cidx_hbm, out_hbm, ridx, cidx, rows0, rows1, acc, sem0, sem1
):
    c = lax.axis_index("c")
    s = lax.axis_index("s")
    _zeros16 = jnp.zeros((16,), jnp.float32)

    def fill_zero(i, _):
        def lane(k, _):
            rows0[i, pl.ds(k * 16, 16)] = _zeros16
            return 0

        lax.fori_loop(0, D // 16, lane, 0)
        return 0

    lax.fori_loop(0, BATCH, fill_zero, 0)

    def zcopy(k, _):
        pltpu.sync_copy(rows0, acc.at[pl.ds(s * RPT + k * BATCH, BATCH), :])
        return 0

    lax.fori_loop(0, RPT // BATCH, zcopy, 0)
    plsc.subcore_barrier()

    rows = (rows0, rows1)
    sems = (sem0, sem1)

    start_blk = jnp.where(c == 0, s * B0, NS * B0 + s * B1)
    nstage = jnp.where(c == 0, B0 // CH, B1 // CH)

    def stage(st, _):
        base = start_blk + st * CH
        pltpu.sync_copy(ridx_hbm.at[pl.ds(base, CH), :], ridx)
        pltpu.sync_copy(cidx_hbm.at[pl.ds(base, CH), :], cidx)

        pltpu.async_copy(g_hbm.at[ridx.at[0]], rows0, sem0)
        pltpu.async_copy(g_hbm.at[ridx.at[1]], rows1, sem1)

        def blk(i, _):
            for b in range(2):
                j = i * 2 + b
                pltpu.make_async_copy(
                    g_hbm.at[ridx.at[j]], rows[b], sems[b]
                ).wait()
                pltpu.sync_copy(rows[b], acc.at[cidx.at[j]], add=True)

                @pl.when(j + 2 < CH)
                def _():
                    pltpu.async_copy(g_hbm.at[ridx.at[j + 2]], rows[b], sems[b])

            return 0

        lax.fori_loop(0, CH // 2, blk, 0)
        return 0

    lax.fori_loop(0, nstage, stage, 0)

    plsc.subcore_barrier()
    pltpu.sync_copy(
        acc.at[pl.ds(s * RPT, RPT), :], out_hbm.at[c, pl.ds(s * RPT, RPT), :]
    )


_seg = pl.kernel(
    _seg_body,
    out_type=jax.ShapeDtypeStruct((NC, NPAD, D), jnp.float32),
    mesh=_mesh,
    scratch_types=[
        pltpu.VMEM((CH, BATCH), jnp.int32),
        pltpu.VMEM((CH, BATCH), jnp.int32),
        pltpu.VMEM((BATCH, D), jnp.float32),
        pltpu.VMEM((BATCH, D), jnp.float32),
        pltpu.VMEM_SHARED((NPAD, D), jnp.float32),
        pltpu.SemaphoreType.DMA,
        pltpu.SemaphoreType.DMA,
    ],
)


BR = 1000


def _dinv(dega_ref, degb_ref):
    return lax.rsqrt(dega_ref[:, 0:1] + degb_ref[:, 0:1] + 1.0)


def _mm_scale_body(dega_ref, degb_ref, x_ref, w_ref, g_ref):
    dinv = _dinv(dega_ref, degb_ref)
    h = jnp.dot(x_ref[...], w_ref[...], preferred_element_type=jnp.float32)
    g_ref[...] = h * dinv


def _mid_body(dega_ref, degb_ref, sa_ref, sb_ref, g_ref, b_ref, w_ref, o_ref):
    dinv = _dinv(dega_ref, degb_ref)
    h = (sa_ref[...] + sb_ref[...] + g_ref[...]) * dinv + b_ref[...]
    h = jnp.maximum(h, 0.0)
    o_ref[...] = jnp.dot(h, w_ref[...], preferred_element_type=jnp.float32) * dinv


def _out_body(dega_ref, degb_ref, sa_ref, sb_ref, g_ref, b_ref, o_ref):
    dinv = _dinv(dega_ref, degb_ref)
    o_ref[...] = jnp.maximum(
        (sa_ref[...] + sb_ref[...] + g_ref[...]) * dinv + b_ref[...], 0.0
    )


_deg_spec = pl.BlockSpec((BR, 1), lambda i: (i, 0))
_row_spec = pl.BlockSpec((BR, D), lambda i: (i, 0))
_w_spec = pl.BlockSpec((D, D), lambda i: (0, 0))
_b_spec = pl.BlockSpec((1, D), lambda i: (0, 0))
_o_shape = jax.ShapeDtypeStruct((N, D), jnp.float32)

_mm_scale = pl.pallas_call(
    _mm_scale_body,
    grid=(N // BR,),
    in_specs=[_deg_spec, _deg_spec, _row_spec, _w_spec],
    out_specs=_row_spec,
    out_shape=_o_shape,
)

_mid = pl.pallas_call(
    _mid_body,
    grid=(N // BR,),
    in_specs=[_deg_spec, _deg_spec, _row_spec, _row_spec, _row_spec, _b_spec, _w_spec],
    out_specs=_row_spec,
    out_shape=_o_shape,
)

_out = pl.pallas_call(
    _out_body,
    grid=(N // BR,),
    in_specs=[_deg_spec, _deg_spec, _row_spec, _row_spec, _row_spec, _b_spec],
    out_specs=_row_spec,
    out_shape=_o_shape,
)


def kernel(x, edge_index, W1, b1, W2, b2):
    row = edge_index[0]
    col = edge_index[1]
    pad = EP - E
    rowp = jnp.concatenate([row, jnp.zeros((pad,), jnp.int32)]).reshape(
        NW, NBLK, BATCH
    )
    colp = jnp.concatenate([col, jnp.full((pad,), N, jnp.int32)]).reshape(
        NW, NBLK, BATCH
    )

    degp = _deg(colp)
    dega = degp[0, :N].reshape(N, 1)
    degb = degp[1, :N].reshape(N, 1)
    b1r = b1.reshape(1, D)
    b2r = b2.reshape(1, D)

    rowf = rowp.reshape(EPB, BATCH)
    colf = colp.reshape(EPB, BATCH)
    g1 = _mm_scale(dega, degb, x, W1)
    s1 = _seg(g1, rowf, colf)
    g2 = _mid(dega, degb, s1[0, :N], s1[1, :N], g1, b1r, W2)
    s2 = _seg(g2, rowf, colf)
    return _out(dega, degb, s2[0, :N], s2[1, :N], g2, b2r)
